# baseline (speedup 1.0000x reference)
import functools

import jax
import jax.numpy as jnp
from jax import lax
from jax.experimental import pallas as pl
from jax.experimental.pallas import tpu as pltpu
from jax.experimental.pallas import tpu_sc as plsc

B = 16384
D = 64
DP = 2 * D
NC = 2
NS = 16
NW = NC * NS
BPW = B // NW
CHUNK = 128
NCHUNK = BPW // CHUNK
NPAIR = 1000000 // 2
NBUF = 4


def _sc_gather2_body(uidx_hbm, iidx_hbm, utab_hbm, itab_hbm, uout_hbm, iout_hbm,
                     uidx_v, iidx_v, *rest):
    bufs = rest[:NBUF]
    sem_g = rest[NBUF:2 * NBUF]
    sem_o = rest[2 * NBUF:3 * NBUF]
    wid = lax.axis_index("s") * NC + lax.axis_index("c")
    base = wid * BPW
    pltpu.sync_copy(uidx_hbm.at[wid], uidx_v)
    pltpu.sync_copy(iidx_hbm.at[wid], iidx_v)

    jobs = [(uidx_v, utab_hbm, uout_hbm, c) for c in range(NCHUNK)]
    jobs += [(iidx_v, itab_hbm, iout_hbm, c) for c in range(NCHUNK)]
    njobs = len(jobs)

    def issue(j):
        idx_v, tab, _, c = jobs[j]
        return pltpu.async_copy(tab.at[idx_v.at[c]], bufs[j % NBUF], sem_g[j % NBUF])

    gh = [issue(j) for j in range(NBUF)]
    oh = [None] * njobs
    for j in range(njobs):
        gh[j].wait()
        _, _, out, c = jobs[j]
        oh[j] = pltpu.async_copy(
            bufs[j % NBUF], out.at[pl.ds(base + c * CHUNK, CHUNK)], sem_o[j % NBUF])
        if j + NBUF < njobs:
            oh[j].wait()
            gh.append(issue(j + NBUF))
    for j in range(njobs - NBUF, njobs):
        oh[j].wait()


@functools.cache
def _sc_gather2():
    mesh = plsc.VectorSubcoreMesh(
        core_axis_name="c", subcore_axis_name="s", num_cores=NC, num_subcores=NS
    )
    scratch = [
        pltpu.VMEM((NCHUNK, CHUNK), jnp.int32),
        pltpu.VMEM((NCHUNK, CHUNK), jnp.int32),
    ]
    scratch += [pltpu.VMEM((CHUNK, DP), jnp.float32) for _ in range(NBUF)]
    scratch += [pltpu.SemaphoreType.DMA for _ in range(2 * NBUF)]
    return pl.kernel(
        _sc_gather2_body,
        out_type=[
            jax.ShapeDtypeStruct((B, DP), jnp.float32),
            jax.ShapeDtypeStruct((B, DP), jnp.float32),
        ],
        mesh=mesh,
        scratch_types=scratch,
    )


BT = 2048


def _mlp_body(xu_ref, xi_ref, pu_ref, pi_ref, w1u_ref, w1i_ref, b1_ref,
              w2_ref, b2_ref, w3_ref, b3_ref, out_ref):
    xu = xu_ref[...]
    xi = xi_ref[...]
    u = jnp.where(pu_ref[...] != 0, xu[:, D:], xu[:, :D])
    i = jnp.where(pi_ref[...] != 0, xi[:, D:], xi[:, :D])
    h = jnp.dot(u, w1u_ref[...], preferred_element_type=jnp.float32)
    h += jnp.dot(i, w1i_ref[...], preferred_element_type=jnp.float32)
    h = jnp.maximum(h + b1_ref[...], 0.0)
    h2 = jnp.dot(h, w2_ref[...], preferred_element_type=jnp.float32)
    h2 = jnp.maximum(h2 + b2_ref[...], 0.0)
    z = jnp.sum(h2 * w3_ref[...], axis=-1) + b3_ref[0, 0]
    out_ref[...] = 1.0 / (1.0 + jnp.exp(-z))


_mlp = pl.pallas_call(
    _mlp_body,
    grid=(B // BT,),
    in_specs=[
        pl.BlockSpec((BT, DP), lambda i: (i, 0)),
        pl.BlockSpec((BT, DP), lambda i: (i, 0)),
        pl.BlockSpec((BT, 1), lambda i: (i, 0)),
        pl.BlockSpec((BT, 1), lambda i: (i, 0)),
        pl.BlockSpec((D, 32), lambda i: (0, 0)),
        pl.BlockSpec((D, 32), lambda i: (0, 0)),
        pl.BlockSpec((1, 32), lambda i: (0, 0)),
        pl.BlockSpec((32, 16), lambda i: (0, 0)),
        pl.BlockSpec((1, 16), lambda i: (0, 0)),
        pl.BlockSpec((1, 16), lambda i: (0, 0)),
        pl.BlockSpec((1, 1), lambda i: (0, 0)),
    ],
    out_specs=pl.BlockSpec((BT,), lambda i: (i,)),
    out_shape=jax.ShapeDtypeStruct((B,), jnp.float32),
)


def kernel(user_indices, item_indices, emb_user, emb_item, W1, b1, W2, b2, W3, b3):
    ui = user_indices.astype(jnp.int32)
    ii = item_indices.astype(jnp.int32)
    upair = (ui >> 1).reshape(NW, NCHUNK, CHUNK)
    ipair = (ii >> 1).reshape(NW, NCHUNK, CHUNK)
    pu = (ui & 1).reshape(B, 1)
    pi = (ii & 1).reshape(B, 1)
    ut2 = emb_user.reshape(NPAIR, DP)
    it2 = emb_item.reshape(NPAIR, DP)
    xu, xi = _sc_gather2()(upair, ipair, ut2, it2)
    w1u = W1[:, :D].T
    w1i = W1[:, D:].T
    return _mlp(xu, xi, pu, pi, w1u, w1i, b1.reshape(1, -1), W2.T,
                b2.reshape(1, -1), W3, b3.reshape(1, 1))


# speedup vs baseline: 1.7421x; 1.7421x over previous
import functools

import jax
import jax.numpy as jnp
from jax import lax
from jax.experimental import pallas as pl
from jax.experimental.pallas import tpu as pltpu
from jax.experimental.pallas import tpu_sc as plsc

B = 16384
D = 64
NC = 2
NS = 16
NW = NC * NS
BPW = B // NW
CHUNK = 128
NCHUNK = BPW // CHUNK
NV = 1000000


DP = 2 * D
NPAIR = NV // 2
NBUF = 4

TW = 32768
TH = TW // 2
NTB = 31
VCUT = NV - NV % 128
TAILW = VCUT - (NTB - 1) * TW
TAILH = TAILW // 2
PBASE = TH * (NTB - 1) + TAILH
NTAIL = NV - VCUT
NPOUT = PBASE + NTAIL


def _tp_eye():
    r = lax.broadcasted_iota(jnp.int32, (D, D), 0)
    c = lax.broadcasted_iota(jnp.int32, (D, D), 1)
    return (r == c).astype(jnp.float32)


def _tp_body(utab, itab, tailu, taili, uout, iout, inb_u, inb_i, outb_u, outb_i,
             sin_u, sin_i, sout_u, sout_i, stail):
    b = pl.program_id(0)
    eye = _tp_eye()

    def in_copy(tab, inb, sem, w):
        return pltpu.make_async_copy(
            tab.at[:, pl.ds(b * TW, w)], inb.at[:, pl.ds(0, w)], sem)

    def out_copy(out, outb, sem, bb, h):
        return pltpu.make_async_copy(
            outb.at[pl.ds(0, h), :], out.at[pl.ds(bb * TH, h), :], sem)

    def transpose_half(a, h):
        left = lax.dot_general(a[:, :h], eye, (((0,), (0,)), ((), ())),
                               preferred_element_type=jnp.float32)
        right = lax.dot_general(a[:, h:2 * h], eye, (((0,), (0,)), ((), ())),
                                preferred_element_type=jnp.float32)
        return jnp.concatenate([left, right], axis=1)

    def step(w, h):
        in_copy(utab, inb_u, sin_u, w).start()
        in_copy(itab, inb_i, sin_i, w).start()

        @pl.when(b > 0)
        def _():
            out_copy(uout, outb_u, sout_u, b - 1, TH).wait()
            out_copy(iout, outb_i, sout_i, b - 1, TH).wait()

        for tab, inb, sin, out, outb, sout in (
                (utab, inb_u, sin_u, uout, outb_u, sout_u),
                (itab, inb_i, sin_i, iout, outb_i, sout_i)):
            in_copy(tab, inb, sin, w).wait()
            outb[pl.ds(0, h), :] = transpose_half(inb[:, pl.ds(0, w)], h)
            out_copy(out, outb, sout, b, h).start()

        @pl.when(b == NTB - 1)
        def _():
            tcu = pltpu.make_async_copy(
                tailu, uout.at[pl.ds(PBASE, NTAIL), :], stail)
            tci = pltpu.make_async_copy(
                taili, iout.at[pl.ds(PBASE, NTAIL), :], stail)
            tcu.start()
            tci.start()
            out_copy(uout, outb_u, sout_u, b, h).wait()
            out_copy(iout, outb_i, sout_i, b, h).wait()
            tcu.wait()
            tci.wait()

    @pl.when(b < NTB - 1)
    def _():
        step(TW, TH)

    @pl.when(b == NTB - 1)
    def _():
        step(TAILW, TAILH)


_tp = pl.pallas_call(
    _tp_body,
    grid=(NTB,),
    in_specs=[
        pl.BlockSpec(memory_space=pl.ANY),
        pl.BlockSpec(memory_space=pl.ANY),
        pl.BlockSpec((NTAIL, DP), lambda i: (0, 0)),
        pl.BlockSpec((NTAIL, DP), lambda i: (0, 0)),
    ],
    out_specs=[
        pl.BlockSpec(memory_space=pl.ANY),
        pl.BlockSpec(memory_space=pl.ANY),
    ],
    out_shape=[
        jax.ShapeDtypeStruct((NPOUT, DP), jnp.float32),
        jax.ShapeDtypeStruct((NPOUT, DP), jnp.float32),
    ],
    scratch_shapes=[
        pltpu.VMEM((D, TW), jnp.float32),
        pltpu.VMEM((D, TW), jnp.float32),
        pltpu.VMEM((TH, DP), jnp.float32),
        pltpu.VMEM((TH, DP), jnp.float32),
        pltpu.SemaphoreType.DMA,
        pltpu.SemaphoreType.DMA,
        pltpu.SemaphoreType.DMA,
        pltpu.SemaphoreType.DMA,
        pltpu.SemaphoreType.DMA,
    ],
)


def _sc_gather2_body(uidx_hbm, iidx_hbm, utab_hbm, itab_hbm, uout_hbm, iout_hbm,
                     uidx_v, iidx_v, *rest):
    bufs = rest[:NBUF]
    sem_g = rest[NBUF:2 * NBUF]
    sem_o = rest[2 * NBUF:3 * NBUF]
    wid = lax.axis_index("s") * NC + lax.axis_index("c")
    base = wid * BPW
    pltpu.sync_copy(uidx_hbm.at[wid], uidx_v)
    pltpu.sync_copy(iidx_hbm.at[wid], iidx_v)

    jobs = [(uidx_v, utab_hbm, uout_hbm, c) for c in range(NCHUNK)]
    jobs += [(iidx_v, itab_hbm, iout_hbm, c) for c in range(NCHUNK)]
    njobs = len(jobs)

    def issue(j):
        idx_v, tab, _, c = jobs[j]
        return pltpu.async_copy(tab.at[idx_v.at[c]], bufs[j % NBUF], sem_g[j % NBUF])

    gh = [issue(j) for j in range(NBUF)]
    oh = [None] * njobs
    for j in range(njobs):
        gh[j].wait()
        _, _, out, c = jobs[j]
        oh[j] = pltpu.async_copy(
            bufs[j % NBUF], out.at[pl.ds(base + c * CHUNK, CHUNK)], sem_o[j % NBUF])
        if j + NBUF < njobs:
            oh[j].wait()
            gh.append(issue(j + NBUF))
    for j in range(njobs - NBUF, njobs):
        oh[j].wait()


@functools.cache
def _sc_gather2():
    mesh = plsc.VectorSubcoreMesh(
        core_axis_name="c", subcore_axis_name="s", num_cores=NC, num_subcores=NS
    )
    scratch = [
        pltpu.VMEM((NCHUNK, CHUNK), jnp.int32),
        pltpu.VMEM((NCHUNK, CHUNK), jnp.int32),
    ]
    scratch += [pltpu.VMEM((CHUNK, DP), jnp.float32) for _ in range(NBUF)]
    scratch += [pltpu.SemaphoreType.DMA for _ in range(2 * NBUF)]
    return pl.kernel(
        _sc_gather2_body,
        out_type=[
            jax.ShapeDtypeStruct((B, DP), jnp.float32),
            jax.ShapeDtypeStruct((B, DP), jnp.float32),
        ],
        mesh=mesh,
        scratch_types=scratch,
    )


BT = 2048


def _mlp_body(xu_ref, xi_ref, pu_ref, pi_ref, w1u_ref, w1i_ref, b1_ref,
              w2_ref, b2_ref, w3_ref, b3_ref, out_ref):
    xu = xu_ref[...]
    xi = xi_ref[...]
    u = jnp.where(pu_ref[...] != 0, xu[:, D:], xu[:, :D])
    i = jnp.where(pi_ref[...] != 0, xi[:, D:], xi[:, :D])
    h = jnp.dot(u, w1u_ref[...], preferred_element_type=jnp.float32)
    h += jnp.dot(i, w1i_ref[...], preferred_element_type=jnp.float32)
    h = jnp.maximum(h + b1_ref[...], 0.0)
    h2 = jnp.dot(h, w2_ref[...], preferred_element_type=jnp.float32)
    h2 = jnp.maximum(h2 + b2_ref[...], 0.0)
    z = jnp.sum(h2 * w3_ref[...], axis=-1) + b3_ref[0, 0]
    out_ref[...] = 1.0 / (1.0 + jnp.exp(-z))


_mlp = pl.pallas_call(
    _mlp_body,
    grid=(B // BT,),
    in_specs=[
        pl.BlockSpec((BT, DP), lambda i: (i, 0)),
        pl.BlockSpec((BT, DP), lambda i: (i, 0)),
        pl.BlockSpec((BT, 1), lambda i: (i, 0)),
        pl.BlockSpec((BT, 1), lambda i: (i, 0)),
        pl.BlockSpec((D, 32), lambda i: (0, 0)),
        pl.BlockSpec((D, 32), lambda i: (0, 0)),
        pl.BlockSpec((1, 32), lambda i: (0, 0)),
        pl.BlockSpec((32, 16), lambda i: (0, 0)),
        pl.BlockSpec((1, 16), lambda i: (0, 0)),
        pl.BlockSpec((1, 16), lambda i: (0, 0)),
        pl.BlockSpec((1, 1), lambda i: (0, 0)),
    ],
    out_specs=pl.BlockSpec((BT,), lambda i: (i,)),
    out_shape=jax.ShapeDtypeStruct((B,), jnp.float32),
)


def kernel(user_indices, item_indices, emb_user, emb_item, W1, b1, W2, b2, W3, b3):
    ui = user_indices.astype(jnp.int32)
    ii = item_indices.astype(jnp.int32)

    def pair_par(idx):
        blk = idx // TW
        w = idx % TW
        h = jnp.where(blk == NTB - 1, TAILH, TH)
        par = ((w >= h) & (idx < VCUT)).astype(jnp.int32)
        pair = blk * TH + w - par * h
        pair = jnp.where(idx >= VCUT, PBASE + (idx - VCUT), pair)
        return pair, par

    upair, pu = pair_par(ui)
    ipair, pi = pair_par(ii)
    upair = upair.reshape(NW, NCHUNK, CHUNK)
    ipair = ipair.reshape(NW, NCHUNK, CHUNK)
    pu = pu.reshape(B, 1)
    pi = pi.reshape(B, 1)
    tail_u = jnp.concatenate([emb_user[VCUT:], emb_user[VCUT:]], axis=1)
    tail_i = jnp.concatenate([emb_item[VCUT:], emb_item[VCUT:]], axis=1)
    ut2, it2 = _tp(emb_user.T, emb_item.T, tail_u, tail_i)
    xu, xi = _sc_gather2()(upair, ipair, ut2, it2)
    w1u = W1[:, :D].T
    w1i = W1[:, D:].T
    return _mlp(xu, xi, pu, pi, w1u, w1i, b1.reshape(1, -1), W2.T,
                b2.reshape(1, -1), W3, b3.reshape(1, 1))


# speedup vs baseline: 1.9135x; 1.0984x over previous
import functools

import jax
import jax.numpy as jnp
from jax import lax
from jax.experimental import pallas as pl
from jax.experimental.pallas import tpu as pltpu
from jax.experimental.pallas import tpu_sc as plsc

B = 16384
D = 64
NC = 2
NS = 16
NW = NC * NS
BPW = B // NW
CHUNK = 128
NCHUNK = BPW // CHUNK
NV = 1000000


DP = 2 * D
NPAIR = NV // 2
NBUF = 4

TW = 16128
TH = TW // 2
VCUT = NV - NV % 128
NTB = VCUT // TW
NPOUT = VCUT // 2
NTAIL = NV - VCUT


def _tp_eye():
    r = lax.broadcasted_iota(jnp.int32, (D, D), 0)
    c = lax.broadcasted_iota(jnp.int32, (D, D), 1)
    return (r == c).astype(jnp.float32)


def _tp_body(u_ref, i_ref, ou_ref, oi_ref):
    eye = _tp_eye()

    def transpose_half(a):
        left = lax.dot_general(a[:, :TH], eye, (((0,), (0,)), ((), ())),
                               preferred_element_type=jnp.float32)
        right = lax.dot_general(a[:, TH:], eye, (((0,), (0,)), ((), ())),
                                preferred_element_type=jnp.float32)
        return jnp.concatenate([left, right], axis=1)

    ou_ref[...] = transpose_half(u_ref[...])
    oi_ref[...] = transpose_half(i_ref[...])


_tp = pl.pallas_call(
    _tp_body,
    grid=(NTB,),
    in_specs=[
        pl.BlockSpec((D, TW), lambda i: (0, i)),
        pl.BlockSpec((D, TW), lambda i: (0, i)),
    ],
    out_specs=[
        pl.BlockSpec((TH, DP), lambda i: (i, 0)),
        pl.BlockSpec((TH, DP), lambda i: (i, 0)),
    ],
    out_shape=[
        jax.ShapeDtypeStruct((NPOUT, DP), jnp.float32),
        jax.ShapeDtypeStruct((NPOUT, DP), jnp.float32),
    ],
)


def _sc_gather2_body(uidx_hbm, iidx_hbm, utab_hbm, itab_hbm, uout_hbm, iout_hbm,
                     uidx_v, iidx_v, *rest):
    bufs = rest[:NBUF]
    sem_g = rest[NBUF:2 * NBUF]
    sem_o = rest[2 * NBUF:3 * NBUF]
    wid = lax.axis_index("s") * NC + lax.axis_index("c")
    base = wid * BPW
    pltpu.sync_copy(uidx_hbm.at[wid], uidx_v)
    pltpu.sync_copy(iidx_hbm.at[wid], iidx_v)

    jobs = [(uidx_v, utab_hbm, uout_hbm, c) for c in range(NCHUNK)]
    jobs += [(iidx_v, itab_hbm, iout_hbm, c) for c in range(NCHUNK)]
    njobs = len(jobs)

    def issue(j):
        idx_v, tab, _, c = jobs[j]
        return pltpu.async_copy(tab.at[idx_v.at[c]], bufs[j % NBUF], sem_g[j % NBUF])

    gh = [issue(j) for j in range(NBUF)]
    oh = [None] * njobs
    for j in range(njobs):
        gh[j].wait()
        _, _, out, c = jobs[j]
        oh[j] = pltpu.async_copy(
            bufs[j % NBUF], out.at[pl.ds(base + c * CHUNK, CHUNK)], sem_o[j % NBUF])
        if j + NBUF < njobs:
            oh[j].wait()
            gh.append(issue(j + NBUF))
    for j in range(njobs - NBUF, njobs):
        oh[j].wait()


@functools.cache
def _sc_gather2():
    mesh = plsc.VectorSubcoreMesh(
        core_axis_name="c", subcore_axis_name="s", num_cores=NC, num_subcores=NS
    )
    scratch = [
        pltpu.VMEM((NCHUNK, CHUNK), jnp.int32),
        pltpu.VMEM((NCHUNK, CHUNK), jnp.int32),
    ]
    scratch += [pltpu.VMEM((CHUNK, DP), jnp.float32) for _ in range(NBUF)]
    scratch += [pltpu.SemaphoreType.DMA for _ in range(2 * NBUF)]
    return pl.kernel(
        _sc_gather2_body,
        out_type=[
            jax.ShapeDtypeStruct((B, DP), jnp.float32),
            jax.ShapeDtypeStruct((B, DP), jnp.float32),
        ],
        mesh=mesh,
        scratch_types=scratch,
    )


BT = 2048


def _mlp_body(xu_ref, xi_ref, pu_ref, pi_ref, w1u_ref, w1i_ref, b1_ref,
              w2_ref, b2_ref, w3_ref, b3_ref, out_ref):
    xu = xu_ref[...]
    xi = xi_ref[...]
    u = jnp.where(pu_ref[...] != 0, xu[:, D:], xu[:, :D])
    i = jnp.where(pi_ref[...] != 0, xi[:, D:], xi[:, :D])
    h = jnp.dot(u, w1u_ref[...], preferred_element_type=jnp.float32)
    h += jnp.dot(i, w1i_ref[...], preferred_element_type=jnp.float32)
    h = jnp.maximum(h + b1_ref[...], 0.0)
    h2 = jnp.dot(h, w2_ref[...], preferred_element_type=jnp.float32)
    h2 = jnp.maximum(h2 + b2_ref[...], 0.0)
    z = jnp.sum(h2 * w3_ref[...], axis=-1) + b3_ref[0, 0]
    out_ref[...] = 1.0 / (1.0 + jnp.exp(-z))


_mlp = pl.pallas_call(
    _mlp_body,
    grid=(B // BT,),
    in_specs=[
        pl.BlockSpec((BT, DP), lambda i: (i, 0)),
        pl.BlockSpec((BT, DP), lambda i: (i, 0)),
        pl.BlockSpec((BT, 1), lambda i: (i, 0)),
        pl.BlockSpec((BT, 1), lambda i: (i, 0)),
        pl.BlockSpec((D, 32), lambda i: (0, 0)),
        pl.BlockSpec((D, 32), lambda i: (0, 0)),
        pl.BlockSpec((1, 32), lambda i: (0, 0)),
        pl.BlockSpec((32, 16), lambda i: (0, 0)),
        pl.BlockSpec((1, 16), lambda i: (0, 0)),
        pl.BlockSpec((1, 16), lambda i: (0, 0)),
        pl.BlockSpec((1, 1), lambda i: (0, 0)),
    ],
    out_specs=pl.BlockSpec((BT,), lambda i: (i,)),
    out_shape=jax.ShapeDtypeStruct((B,), jnp.float32),
)


def kernel(user_indices, item_indices, emb_user, emb_item, W1, b1, W2, b2, W3, b3):
    ui = user_indices.astype(jnp.int32)
    ii = item_indices.astype(jnp.int32)

    def pair_par(idx):
        blk = idx // TW
        w = idx % TW
        par = ((w >= TH) & (idx < VCUT)).astype(jnp.int32)
        pair = blk * TH + w - par * TH
        pair = jnp.where(idx >= VCUT, 0, pair)
        return pair, par

    upair, pu = pair_par(ui)
    ipair, pi = pair_par(ii)
    upair = upair.reshape(NW, NCHUNK, CHUNK)
    ipair = ipair.reshape(NW, NCHUNK, CHUNK)
    pu = pu.reshape(B, 1)
    pi = pi.reshape(B, 1)
    ut2, it2 = _tp(emb_user.T, emb_item.T)
    xu, xi = _sc_gather2()(upair, ipair, ut2, it2)
    tail_u = jnp.concatenate([emb_user[VCUT:], emb_user[VCUT:]], axis=1)
    tail_i = jnp.concatenate([emb_item[VCUT:], emb_item[VCUT:]], axis=1)
    xu = jnp.where((ui >= VCUT)[:, None],
                   jnp.take(tail_u, jnp.clip(ui - VCUT, 0, NTAIL - 1), axis=0), xu)
    xi = jnp.where((ii >= VCUT)[:, None],
                   jnp.take(tail_i, jnp.clip(ii - VCUT, 0, NTAIL - 1), axis=0), xi)
    w1u = W1[:, :D].T
    w1i = W1[:, D:].T
    return _mlp(xu, xi, pu, pi, w1u, w1i, b1.reshape(1, -1), W2.T,
                b2.reshape(1, -1), W3, b3.reshape(1, 1))


# speedup vs baseline: 2.3288x; 1.2170x over previous
import functools

import jax
import jax.numpy as jnp
from jax import lax
from jax.experimental import pallas as pl
from jax.experimental.pallas import tpu as pltpu
from jax.experimental.pallas import tpu_sc as plsc

B = 16384
D = 64
NC = 2
NS = 16
NW = NC * NS
BPW = B // NW
CHUNK = 128
NCHUNK = BPW // CHUNK
NV = 1000000


DP = 2 * D
NPAIR = NV // 2
NBUF = 4

TW = 16128
TH = TW // 2
VCUT = NV - NV % 128
NTB = VCUT // TW
NPOUT = VCUT // 2
NTAIL = NV - VCUT


def _tp_eye():
    r = lax.broadcasted_iota(jnp.int32, (D, D), 0)
    c = lax.broadcasted_iota(jnp.int32, (D, D), 1)
    return (r == c).astype(jnp.float32)


def _tp_body(u_ref, i_ref, ou_ref, oi_ref):
    eye = _tp_eye()

    def transpose_half(a):
        left = lax.dot_general(a[:, :TH], eye, (((0,), (0,)), ((), ())),
                               preferred_element_type=jnp.float32)
        right = lax.dot_general(a[:, TH:], eye, (((0,), (0,)), ((), ())),
                                preferred_element_type=jnp.float32)
        return jnp.concatenate([left, right], axis=1)

    ou_ref[...] = transpose_half(u_ref[...])
    oi_ref[...] = transpose_half(i_ref[...])


_tp = pl.pallas_call(
    _tp_body,
    grid=(NTB,),
    in_specs=[
        pl.BlockSpec((D, TW), lambda i: (0, i)),
        pl.BlockSpec((D, TW), lambda i: (0, i)),
    ],
    out_specs=[
        pl.BlockSpec((TH, DP), lambda i: (i, 0)),
        pl.BlockSpec((TH, DP), lambda i: (i, 0)),
    ],
    out_shape=[
        jax.ShapeDtypeStruct((NPOUT, DP), jnp.float32),
        jax.ShapeDtypeStruct((NPOUT, DP), jnp.float32),
    ],
)


def _sc_gather2_body(uidx_hbm, iidx_hbm, utab_hbm, itab_hbm, uout_hbm, iout_hbm,
                     uidx_v, iidx_v, *rest):
    bufs = rest[:NBUF]
    sem_g = rest[NBUF:2 * NBUF]
    sem_o = rest[2 * NBUF:3 * NBUF]
    wid = lax.axis_index("s") * NC + lax.axis_index("c")
    base = wid * BPW
    pltpu.sync_copy(uidx_hbm.at[wid], uidx_v)
    pltpu.sync_copy(iidx_hbm.at[wid], iidx_v)

    jobs = [(uidx_v, utab_hbm, uout_hbm, c) for c in range(NCHUNK)]
    jobs += [(iidx_v, itab_hbm, iout_hbm, c) for c in range(NCHUNK)]
    njobs = len(jobs)

    def issue(j):
        idx_v, tab, _, c = jobs[j]
        return pltpu.async_copy(tab.at[idx_v.at[c]], bufs[j % NBUF], sem_g[j % NBUF])

    gh = [issue(j) for j in range(NBUF)]
    oh = [None] * njobs
    for j in range(njobs):
        gh[j].wait()
        _, _, out, c = jobs[j]
        oh[j] = pltpu.async_copy(
            bufs[j % NBUF], out.at[pl.ds(base + c * CHUNK, CHUNK)], sem_o[j % NBUF])
        if j + NBUF < njobs:
            oh[j].wait()
            gh.append(issue(j + NBUF))
    for j in range(njobs - NBUF, njobs):
        oh[j].wait()


@functools.cache
def _sc_gather2():
    mesh = plsc.VectorSubcoreMesh(
        core_axis_name="c", subcore_axis_name="s", num_cores=NC, num_subcores=NS
    )
    scratch = [
        pltpu.VMEM((NCHUNK, CHUNK), jnp.int32),
        pltpu.VMEM((NCHUNK, CHUNK), jnp.int32),
    ]
    scratch += [pltpu.VMEM((CHUNK, DP), jnp.float32) for _ in range(NBUF)]
    scratch += [pltpu.SemaphoreType.DMA for _ in range(2 * NBUF)]
    return pl.kernel(
        _sc_gather2_body,
        out_type=[
            jax.ShapeDtypeStruct((B, DP), jnp.float32),
            jax.ShapeDtypeStruct((B, DP), jnp.float32),
        ],
        mesh=mesh,
        scratch_types=scratch,
    )


BT = 2048


def _mlp_body(xu_ref, xi_ref, pu_ref, pi_ref, ohu_ref, ohi_ref,
              tailu_ref, taili_ref, w1u_ref, w1i_ref, b1_ref,
              w2_ref, b2_ref, w3_ref, b3_ref, out_ref):
    xu = xu_ref[...]
    xi = xi_ref[...]
    u = jnp.where(pu_ref[...] > 0, xu[:, D:], xu[:, :D])
    i = jnp.where(pi_ref[...] > 0, xi[:, D:], xi[:, :D])
    u = jnp.where(pu_ref[...] < 0,
                  jnp.dot(ohu_ref[...], tailu_ref[...],
                          preferred_element_type=jnp.float32), u)
    i = jnp.where(pi_ref[...] < 0,
                  jnp.dot(ohi_ref[...], taili_ref[...],
                          preferred_element_type=jnp.float32), i)
    h = jnp.dot(u, w1u_ref[...], preferred_element_type=jnp.float32)
    h += jnp.dot(i, w1i_ref[...], preferred_element_type=jnp.float32)
    h = jnp.maximum(h + b1_ref[...], 0.0)
    h2 = jnp.dot(h, w2_ref[...], preferred_element_type=jnp.float32)
    h2 = jnp.maximum(h2 + b2_ref[...], 0.0)
    z = jnp.sum(h2 * w3_ref[...], axis=-1) + b3_ref[0, 0]
    out_ref[...] = 1.0 / (1.0 + jnp.exp(-z))


_mlp = pl.pallas_call(
    _mlp_body,
    grid=(B // BT,),
    in_specs=[
        pl.BlockSpec((BT, DP), lambda i: (i, 0)),
        pl.BlockSpec((BT, DP), lambda i: (i, 0)),
        pl.BlockSpec((BT, 1), lambda i: (i, 0)),
        pl.BlockSpec((BT, 1), lambda i: (i, 0)),
        pl.BlockSpec((BT, D), lambda i: (i, 0)),
        pl.BlockSpec((BT, D), lambda i: (i, 0)),
        pl.BlockSpec((NTAIL, D), lambda i: (0, 0)),
        pl.BlockSpec((NTAIL, D), lambda i: (0, 0)),
        pl.BlockSpec((D, 32), lambda i: (0, 0)),
        pl.BlockSpec((D, 32), lambda i: (0, 0)),
        pl.BlockSpec((1, 32), lambda i: (0, 0)),
        pl.BlockSpec((32, 16), lambda i: (0, 0)),
        pl.BlockSpec((1, 16), lambda i: (0, 0)),
        pl.BlockSpec((1, 16), lambda i: (0, 0)),
        pl.BlockSpec((1, 1), lambda i: (0, 0)),
    ],
    out_specs=pl.BlockSpec((BT,), lambda i: (i,)),
    out_shape=jax.ShapeDtypeStruct((B,), jnp.float32),
)


def kernel(user_indices, item_indices, emb_user, emb_item, W1, b1, W2, b2, W3, b3):
    ui = user_indices.astype(jnp.int32)
    ii = item_indices.astype(jnp.int32)

    def pair_par(idx):
        blk = idx // TW
        w = idx % TW
        par = ((w >= TH) & (idx < VCUT)).astype(jnp.int32)
        pair = blk * TH + w - par * TH
        pair = jnp.where(idx >= VCUT, 0, pair)
        par = jnp.where(idx >= VCUT, -1, par)
        return pair, par

    upair, pu = pair_par(ui)
    ipair, pi = pair_par(ii)
    upair = upair.reshape(NW, NCHUNK, CHUNK)
    ipair = ipair.reshape(NW, NCHUNK, CHUNK)
    pu = pu.reshape(B, 1)
    pi = pi.reshape(B, 1)
    ut2, it2 = _tp(emb_user.T, emb_item.T)
    xu, xi = _sc_gather2()(upair, ipair, ut2, it2)
    ohu = jax.nn.one_hot(ui - VCUT, NTAIL, dtype=jnp.float32)
    ohi = jax.nn.one_hot(ii - VCUT, NTAIL, dtype=jnp.float32)
    w1u = W1[:, :D].T
    w1i = W1[:, D:].T
    return _mlp(xu, xi, pu, pi, ohu, ohi, emb_user[VCUT:], emb_item[VCUT:],
                w1u, w1i, b1.reshape(1, -1), W2.T,
                b2.reshape(1, -1), W3, b3.reshape(1, 1))
